# Initial kernel scaffold; baseline (speedup 1.0000x reference)
import jax
import jax.numpy as jnp
from jax.experimental import pallas as pl


def kernel(x, grid):
    raise NotImplementedError("write your pallas kernel here")


# speedup vs baseline: 1.0230x; 1.0230x over previous
import functools

import jax
import jax.numpy as jnp
from jax import lax
from jax.experimental import pallas as pl
from jax.experimental.pallas import tpu as pltpu
from jax.experimental.pallas import tpu_sc as plsc

N, C, H, W = 4, 96, 224, 224
HW = H * W
L = 16
NC, NS = 2, 16
CPAIRS = C // 2
TASKS_PER_TILE = (2 * CPAIRS) // NS
CHUNK_ROWS = 16
ROW_VECS = W // L
CHUNK = CHUNK_ROWS * W
NCHUNKS = H // CHUNK_ROWS
SX = float(W) / float(W - 1)
SY = float(H) / float(H - 1)


def _floor_i32(p):
    t = p.astype(jnp.int32)
    tf = t.astype(jnp.float32)
    return jnp.where(tf > p, t - 1, t)


def _sc_body(x_hbm, gx_hbm, gy_hbm, out_hbm, img_v, gx_v, gy_v, o0_v, o1_v):
    cid = lax.axis_index("c")
    sid = lax.axis_index("s")

    def task_fn(t, carry):
        tt = sid * TASKS_PER_TILE + t
        n_local = tt // CPAIRS
        cp = tt % CPAIRS
        n = cid * 2 + n_local
        chan = n * C + cp * 2
        pltpu.sync_copy(x_hbm.at[pl.ds(chan * HW, 2 * HW)], img_v)
        goff = n * HW

        def chunk_fn(r16, carry2):
            roff = goff + r16 * CHUNK
            pltpu.sync_copy(gx_hbm.at[pl.ds(roff, CHUNK)], gx_v)
            pltpu.sync_copy(gy_hbm.at[pl.ds(roff, CHUNK)], gy_v)

            def row_fn(ri, carry3):
                i = r16 * CHUNK_ROWS + ri
                i_f = lax.convert_element_type(i, jnp.float32)

                def vec_fn(v, carry4):
                    off = (ri * ROW_VECS + v) * L
                    gxv = gx_v[pl.ds(off, L)]
                    gyv = gy_v[pl.ds(off, L)]
                    jb = lax.convert_element_type(v * L, jnp.float32)
                    jf = lax.iota(jnp.int32, L).astype(jnp.float32) + jb
                    px = (gxv + jf) * SX - 0.5
                    py = (gyv + i_f) * SY - 0.5
                    px = jnp.minimum(jnp.maximum(px, -2.0), float(W + 2))
                    py = jnp.minimum(jnp.maximum(py, -2.0), float(H + 2))
                    x0 = _floor_i32(px)
                    y0 = _floor_i32(py)
                    x0f = x0.astype(jnp.float32)
                    y0f = y0.astype(jnp.float32)
                    wx1 = px - x0f
                    wx0 = 1.0 - wx1
                    wy1 = py - y0f
                    wy0 = 1.0 - wy1
                    zero = jnp.zeros((L,), jnp.float32)
                    wx0 = jnp.where((x0 >= 0) & (x0 <= W - 1), wx0, zero)
                    wx1 = jnp.where((x0 >= -1) & (x0 <= W - 2), wx1, zero)
                    wy0 = jnp.where((y0 >= 0) & (y0 <= H - 1), wy0, zero)
                    wy1 = jnp.where((y0 >= -1) & (y0 <= H - 2), wy1, zero)
                    w00 = wx0 * wy0
                    w01 = wx1 * wy0
                    w10 = wx0 * wy1
                    w11 = wx1 * wy1
                    xc0 = jnp.maximum(jnp.minimum(x0, W - 1), 0)
                    xc1 = jnp.maximum(jnp.minimum(x0 + 1, W - 1), 0)
                    yb0 = jnp.maximum(jnp.minimum(y0, H - 1), 0) * W
                    yb1 = jnp.maximum(jnp.minimum(y0 + 1, H - 1), 0) * W
                    i00 = yb0 + xc0
                    i01 = yb0 + xc1
                    i10 = yb1 + xc0
                    i11 = yb1 + xc1
                    a00 = plsc.load_gather(img_v, [i00])
                    a01 = plsc.load_gather(img_v, [i01])
                    a10 = plsc.load_gather(img_v, [i10])
                    a11 = plsc.load_gather(img_v, [i11])
                    o0_v[pl.ds(off, L)] = (w00 * a00 + w01 * a01
                                           + w10 * a10 + w11 * a11)
                    hwv = jnp.full((L,), HW, jnp.int32)
                    b00 = plsc.load_gather(img_v, [i00 + hwv])
                    b01 = plsc.load_gather(img_v, [i01 + hwv])
                    b10 = plsc.load_gather(img_v, [i10 + hwv])
                    b11 = plsc.load_gather(img_v, [i11 + hwv])
                    o1_v[pl.ds(off, L)] = (w00 * b00 + w01 * b01
                                           + w10 * b10 + w11 * b11)
                    return carry4

                return lax.fori_loop(0, ROW_VECS, vec_fn, carry3)

            lax.fori_loop(0, CHUNK_ROWS, row_fn, carry2)
            obase = r16 * CHUNK
            pltpu.sync_copy(o0_v, out_hbm.at[pl.ds(chan * HW + obase, CHUNK)])
            pltpu.sync_copy(o1_v,
                            out_hbm.at[pl.ds((chan + 1) * HW + obase, CHUNK)])
            return carry2

        return lax.fori_loop(0, NCHUNKS, chunk_fn, carry)

    lax.fori_loop(0, TASKS_PER_TILE, task_fn, 0)


@jax.jit
def _grid_sample_sc(xf, gx, gy):
    mesh = plsc.VectorSubcoreMesh(core_axis_name="c", subcore_axis_name="s",
                                  num_cores=NC, num_subcores=NS)
    return pl.kernel(
        _sc_body,
        out_type=jax.ShapeDtypeStruct((N * C * HW,), jnp.float32),
        mesh=mesh,
        compiler_params=pltpu.CompilerParams(needs_layout_passes=False),
        scratch_types=[
            pltpu.VMEM((2 * HW,), jnp.float32),
            pltpu.VMEM((CHUNK,), jnp.float32),
            pltpu.VMEM((CHUNK,), jnp.float32),
            pltpu.VMEM((CHUNK,), jnp.float32),
            pltpu.VMEM((CHUNK,), jnp.float32),
        ],
    )(xf, gx, gy)


def kernel(x, grid):
    xf = x.reshape(N * C * HW)
    gx = grid[..., 0].reshape(N * HW)
    gy = grid[..., 1].reshape(N * HW)
    out = _grid_sample_sc(xf, gx, gy)
    return out.reshape(N, C, H, W)


# speedup vs baseline: 1.7669x; 1.7272x over previous
import jax
import jax.numpy as jnp
from jax import lax
from jax.experimental import pallas as pl
from jax.experimental.pallas import tpu as pltpu
from jax.experimental.pallas import tpu_sc as plsc

N, C, H, W = 4, 96, 224, 224
HW = H * W
L = 16
NC, NS = 2, 16
CPAIRS = C // 2
TASKS_PER_TILE = (2 * CPAIRS) // NS
CHUNK_ROWS = 14
ROW_VECS = W // L
CHUNK = CHUNK_ROWS * W
NCHUNKS = H // CHUNK_ROWS
SX = float(W) / float(W - 1)
SY = float(H) / float(H - 1)


def _floor_i32(p):
    t = p.astype(jnp.int32)
    tf = t.astype(jnp.float32)
    return jnp.where(tf > p, t - 1, t)


def _sc_body(x_hbm, gxy_hbm, out_hbm, img_v, g_v, o_v,
             sga, sgb, soa, sob):
    cid = lax.axis_index("c")
    sid = lax.axis_index("s")

    def g_src(n, ck):
        return gxy_hbm.at[pl.ds((n * NCHUNKS + ck) * 2, 2)]

    def compute_chunk(buf, ck, chan_base):
        def row_fn(ri, carry):
            i = ck * CHUNK_ROWS + ri
            i_f = lax.convert_element_type(i, jnp.float32)

            @plsc.parallel_loop(0, ROW_VECS, unroll=2)
            def vec_fn(v):
                off = ri * W + v * L
                gxv = g_v[buf, 0, pl.ds(off, L)]
                gyv = g_v[buf, 1, pl.ds(off, L)]
                jb = lax.convert_element_type(v * L, jnp.float32)
                jf = lax.iota(jnp.int32, L).astype(jnp.float32) + jb
                px = (gxv + jf) * SX - 0.5
                py = (gyv + i_f) * SY - 0.5
                px = jnp.minimum(jnp.maximum(px, -2.0), float(W + 2))
                py = jnp.minimum(jnp.maximum(py, -2.0), float(H + 2))
                x0 = _floor_i32(px)
                y0 = _floor_i32(py)
                wx1 = px - x0.astype(jnp.float32)
                wx0 = 1.0 - wx1
                wy1 = py - y0.astype(jnp.float32)
                wy0 = 1.0 - wy1
                zero = jnp.zeros((L,), jnp.float32)
                wx0 = jnp.where((x0 >= 0) & (x0 <= W - 1), wx0, zero)
                wx1 = jnp.where((x0 >= -1) & (x0 <= W - 2), wx1, zero)
                wy0 = jnp.where((y0 >= 0) & (y0 <= H - 1), wy0, zero)
                wy1 = jnp.where((y0 >= -1) & (y0 <= H - 2), wy1, zero)
                w00 = wx0 * wy0
                w01 = wx1 * wy0
                w10 = wx0 * wy1
                w11 = wx1 * wy1
                xc0 = jnp.maximum(jnp.minimum(x0, W - 1), 0)
                xc1 = jnp.maximum(jnp.minimum(x0 + 1, W - 1), 0)
                yb0 = jnp.maximum(jnp.minimum(y0, H - 1), 0) * W
                yb1 = jnp.maximum(jnp.minimum(y0 + 1, H - 1), 0) * W
                i00 = yb0 + xc0
                i01 = yb0 + xc1
                i10 = yb1 + xc0
                i11 = yb1 + xc1
                a00 = plsc.load_gather(img_v, [i00])
                a01 = plsc.load_gather(img_v, [i01])
                a10 = plsc.load_gather(img_v, [i10])
                a11 = plsc.load_gather(img_v, [i11])
                o_v[buf, 0, 0, pl.ds(off, L)] = (w00 * a00 + w01 * a01
                                                 + w10 * a10 + w11 * a11)
                hwv = jnp.full((L,), HW, jnp.int32)
                b00 = plsc.load_gather(img_v, [i00 + hwv])
                b01 = plsc.load_gather(img_v, [i01 + hwv])
                b10 = plsc.load_gather(img_v, [i10 + hwv])
                b11 = plsc.load_gather(img_v, [i11 + hwv])
                o_v[buf, 1, 0, pl.ds(off, L)] = (w00 * b00 + w01 * b01
                                                 + w10 * b10 + w11 * b11)

            return carry

        lax.fori_loop(0, CHUNK_ROWS, row_fn, 0)

    def task_fn(t, carry):
        tt = sid * TASKS_PER_TILE + t
        n_local = tt // CPAIRS
        cp = tt % CPAIRS
        n = cid * 2 + n_local
        chan = n * C + cp * 2
        pltpu.sync_copy(x_hbm.at[pl.ds(chan * HW, 2 * HW)], img_v)
        pltpu.async_copy(g_src(n, 0), g_v.at[0], sga)

        def out_dst(ck):
            return out_hbm.at[pl.ds(chan, 2), pl.ds(ck, 1)]

        def half(k, buf, sg_this, sg_next, so_this):
            ck = k * 2 + buf
            nxt = jnp.minimum(ck + 1, NCHUNKS - 1)
            pltpu.async_copy(g_src(n, nxt), g_v.at[1 - buf], sg_next)
            pltpu.make_async_copy(g_src(n, ck), g_v.at[buf], sg_this).wait()

            @pl.when(k > 0)
            def _():
                pltpu.make_async_copy(o_v.at[buf], out_dst(ck),
                                      so_this).wait()

            compute_chunk(buf, ck, chan)
            pltpu.async_copy(o_v.at[buf], out_dst(ck), so_this)

        def chunk_pair(k, carry2):
            half(k, 0, sga, sgb, soa)
            half(k, 1, sgb, sga, sob)
            return carry2

        lax.fori_loop(0, NCHUNKS // 2, chunk_pair, 0)
        pltpu.make_async_copy(g_src(n, NCHUNKS - 1), g_v.at[0], sga).wait()
        pltpu.make_async_copy(o_v.at[0], out_dst(NCHUNKS - 2), soa).wait()
        pltpu.make_async_copy(o_v.at[1], out_dst(NCHUNKS - 1), sob).wait()
        return carry

    lax.fori_loop(0, TASKS_PER_TILE, task_fn, 0)


@jax.jit
def _grid_sample_sc(xf, gxy):
    mesh = plsc.VectorSubcoreMesh(core_axis_name="c", subcore_axis_name="s",
                                  num_cores=NC, num_subcores=NS)
    return pl.kernel(
        _sc_body,
        out_type=jax.ShapeDtypeStruct((N * C, NCHUNKS, CHUNK), jnp.float32),
        mesh=mesh,
        compiler_params=pltpu.CompilerParams(needs_layout_passes=False),
        scratch_types=[
            pltpu.VMEM((2 * HW,), jnp.float32),
            pltpu.VMEM((2, 2, CHUNK), jnp.float32),
            pltpu.VMEM((2, 2, 1, CHUNK), jnp.float32),
            pltpu.SemaphoreType.DMA,
            pltpu.SemaphoreType.DMA,
            pltpu.SemaphoreType.DMA,
            pltpu.SemaphoreType.DMA,
        ],
    )(xf, gxy)


def kernel(x, grid):
    xf = x.reshape(N * C * HW)
    gxy = jnp.stack([grid[..., 0].reshape(N, NCHUNKS, CHUNK),
                     grid[..., 1].reshape(N, NCHUNKS, CHUNK)],
                    axis=2).reshape(N * NCHUNKS * 2, CHUNK)
    out = _grid_sample_sc(xf, gxy)
    return out.reshape(N, C, H, W)


# speedup vs baseline: 2.3905x; 1.3529x over previous
import jax
import jax.numpy as jnp
from jax import lax
from jax.experimental import pallas as pl
from jax.experimental.pallas import tpu as pltpu
from jax.experimental.pallas import tpu_sc as plsc

N, C, H, W = 4, 96, 224, 224
HW = H * W
L = 16
NC, NS = 2, 16
NW = NC * NS
CPAIRS = C // 2
TASKS_PER_TILE = (2 * CPAIRS) // NS
CHUNK_ROWS = 7
ROW_VECS = W // L
CHUNK = CHUNK_ROWS * W
CVECS = CHUNK // L
NCHUNKS = H // CHUNK_ROWS
JOBS_PER_TILE = (N * NCHUNKS) // NW
SX = float(W) / float(W - 1)
SY = float(H) / float(H - 1)


def _floor_i32(p):
    t = p.astype(jnp.int32)
    tf = t.astype(jnp.float32)
    return jnp.where(tf > p, t - 1, t)


def _tables_body(gxy_hbm, tbl_hbm, g_v, st_v):
    cid = lax.axis_index("c")
    sid = lax.axis_index("s")
    wid = sid * NC + cid

    def job_fn(q, carry):
        g = wid * JOBS_PER_TILE + q
        n = g // NCHUNKS
        ck = g % NCHUNKS
        pltpu.sync_copy(
            gxy_hbm.at[pl.ds((n * NCHUNKS + ck) * 2 * CHUNK, 2 * CHUNK)],
            g_v)

        def row_fn(ri, carry2):
            i = ck * CHUNK_ROWS + ri
            i_f = lax.convert_element_type(i, jnp.float32)

            @plsc.parallel_loop(0, ROW_VECS, unroll=2)
            def vec_fn(v):
                off = ri * W + v * L
                gxv = g_v[pl.ds(off, L)]
                gyv = g_v[pl.ds(CHUNK + off, L)]
                jb = lax.convert_element_type(v * L, jnp.float32)
                jf = lax.iota(jnp.int32, L).astype(jnp.float32) + jb
                px = (gxv + jf) * SX - 0.5
                py = (gyv + i_f) * SY - 0.5
                px = jnp.minimum(jnp.maximum(px, -2.0), float(W + 2))
                py = jnp.minimum(jnp.maximum(py, -2.0), float(H + 2))
                x0 = _floor_i32(px)
                y0 = _floor_i32(py)
                wx1 = px - x0.astype(jnp.float32)
                wx0 = 1.0 - wx1
                wy1 = py - y0.astype(jnp.float32)
                wy0 = 1.0 - wy1
                zero = jnp.zeros((L,), jnp.float32)
                wx0 = jnp.where((x0 >= 0) & (x0 <= W - 1), wx0, zero)
                wx1 = jnp.where((x0 >= -1) & (x0 <= W - 2), wx1, zero)
                wy0 = jnp.where((y0 >= 0) & (y0 <= H - 1), wy0, zero)
                wy1 = jnp.where((y0 >= -1) & (y0 <= H - 2), wy1, zero)
                xc0 = jnp.maximum(jnp.minimum(x0, W - 1), 0)
                xc1 = jnp.maximum(jnp.minimum(x0 + 1, W - 1), 0)
                yb0 = jnp.maximum(jnp.minimum(y0, H - 1), 0) * W
                yb1 = jnp.maximum(jnp.minimum(y0 + 1, H - 1), 0) * W
                i00 = yb0 + xc0
                dpack = (xc1 - xc0) + lax.shift_left(yb1 - yb0, 16)
                st_v[pl.ds(0 * CHUNK + off, L)] = plsc.bitcast(
                    i00, jnp.float32)
                st_v[pl.ds(1 * CHUNK + off, L)] = plsc.bitcast(
                    dpack, jnp.float32)
                st_v[pl.ds(2 * CHUNK + off, L)] = wx0
                st_v[pl.ds(3 * CHUNK + off, L)] = wx1
                st_v[pl.ds(4 * CHUNK + off, L)] = wy0
                st_v[pl.ds(5 * CHUNK + off, L)] = wy1

            return carry2

        lax.fori_loop(0, CHUNK_ROWS, row_fn, 0)
        pltpu.sync_copy(
            st_v,
            tbl_hbm.at[pl.ds((n * NCHUNKS + ck) * 6 * CHUNK, 6 * CHUNK)])
        return carry

    lax.fori_loop(0, JOBS_PER_TILE, job_fn, 0)


def _sample_body(x_hbm, tbl_hbm, out_hbm, img_v, ta_v, tb_v,
                 oa0_v, oa1_v, ob0_v, ob1_v, sta, stb, soa, sob):
    cid = lax.axis_index("c")
    sid = lax.axis_index("s")

    def t_src(n, ck):
        return tbl_hbm.at[pl.ds((n * NCHUNKS + ck) * 6 * CHUNK, 6 * CHUNK)]

    def compute_chunk(t_v, o0_v, o1_v):
        @plsc.parallel_loop(0, CVECS, unroll=2)
        def vec_fn(p):
            off = p * L
            i00 = plsc.bitcast(t_v[pl.ds(0 * CHUNK + off, L)], jnp.int32)
            dpk = plsc.bitcast(t_v[pl.ds(1 * CHUNK + off, L)], jnp.int32)
            wx0 = t_v[pl.ds(2 * CHUNK + off, L)]
            wx1 = t_v[pl.ds(3 * CHUNK + off, L)]
            wy0 = t_v[pl.ds(4 * CHUNK + off, L)]
            wy1 = t_v[pl.ds(5 * CHUNK + off, L)]
            dx = dpk & 0xFFFF
            dyw = lax.shift_right_arithmetic(dpk, 16)
            i01 = i00 + dx
            i10 = i00 + dyw
            i11 = i01 + dyw
            w00 = wx0 * wy0
            w01 = wx1 * wy0
            w10 = wx0 * wy1
            w11 = wx1 * wy1
            a00 = plsc.load_gather(img_v, [i00])
            a01 = plsc.load_gather(img_v, [i01])
            a10 = plsc.load_gather(img_v, [i10])
            a11 = plsc.load_gather(img_v, [i11])
            o0_v[pl.ds(off, L)] = (w00 * a00 + w01 * a01
                                   + w10 * a10 + w11 * a11)
            hwv = jnp.full((L,), HW, jnp.int32)
            b00 = plsc.load_gather(img_v, [i00 + hwv])
            b01 = plsc.load_gather(img_v, [i01 + hwv])
            b10 = plsc.load_gather(img_v, [i10 + hwv])
            b11 = plsc.load_gather(img_v, [i11 + hwv])
            o1_v[pl.ds(off, L)] = (w00 * b00 + w01 * b01
                                   + w10 * b10 + w11 * b11)

    def task_fn(t, carry):
        tt = sid * TASKS_PER_TILE + t
        n_local = tt // CPAIRS
        cp = tt % CPAIRS
        n = cid * 2 + n_local
        chan = n * C + cp * 2
        pltpu.sync_copy(x_hbm.at[pl.ds(chan * HW, 2 * HW)], img_v)
        pltpu.async_copy(t_src(n, 0), ta_v, sta)

        def out_dst(ck, ch):
            return out_hbm.at[pl.ds((chan + ch) * HW + ck * CHUNK, CHUNK)]

        def half(k, buf, t_v, t_next, o0_v, o1_v, st_this, st_next,
                 so_this):
            ck = k * 2 + buf
            nxt = jnp.minimum(ck + 1, NCHUNKS - 1)
            pltpu.async_copy(t_src(n, nxt), t_next, st_next)
            pltpu.make_async_copy(t_src(n, ck), t_v, st_this).wait()

            @pl.when(k > 0)
            def _():
                pltpu.make_async_copy(o0_v, out_dst(ck, 0), so_this).wait()
                pltpu.make_async_copy(o1_v, out_dst(ck, 1), so_this).wait()

            compute_chunk(t_v, o0_v, o1_v)
            pltpu.async_copy(o0_v, out_dst(ck, 0), so_this)
            pltpu.async_copy(o1_v, out_dst(ck, 1), so_this)

        def chunk_pair(k, carry2):
            half(k, 0, ta_v, tb_v, oa0_v, oa1_v, sta, stb, soa)
            half(k, 1, tb_v, ta_v, ob0_v, ob1_v, stb, sta, sob)
            return carry2

        lax.fori_loop(0, NCHUNKS // 2, chunk_pair, 0)
        pltpu.make_async_copy(t_src(n, NCHUNKS - 1), ta_v, sta).wait()
        pltpu.make_async_copy(oa0_v, out_dst(NCHUNKS - 2, 0), soa).wait()
        pltpu.make_async_copy(oa1_v, out_dst(NCHUNKS - 2, 1), soa).wait()
        pltpu.make_async_copy(ob0_v, out_dst(NCHUNKS - 1, 0), sob).wait()
        pltpu.make_async_copy(ob1_v, out_dst(NCHUNKS - 1, 1), sob).wait()
        return carry

    lax.fori_loop(0, TASKS_PER_TILE, task_fn, 0)


@jax.jit
def _grid_sample_sc(xf, gxy):
    mesh = plsc.VectorSubcoreMesh(core_axis_name="c", subcore_axis_name="s",
                                  num_cores=NC, num_subcores=NS)
    tbl = pl.kernel(
        _tables_body,
        out_type=jax.ShapeDtypeStruct((N * NCHUNKS * 6 * CHUNK,),
                                      jnp.float32),
        mesh=mesh,
        compiler_params=pltpu.CompilerParams(needs_layout_passes=False),
        scratch_types=[
            pltpu.VMEM((2 * CHUNK,), jnp.float32),
            pltpu.VMEM((6 * CHUNK,), jnp.float32),
        ],
    )(gxy)
    return pl.kernel(
        _sample_body,
        out_type=jax.ShapeDtypeStruct((N * C * HW,), jnp.float32),
        mesh=mesh,
        compiler_params=pltpu.CompilerParams(needs_layout_passes=False),
        scratch_types=[
            pltpu.VMEM((2 * HW,), jnp.float32),
            pltpu.VMEM((6 * CHUNK,), jnp.float32),
            pltpu.VMEM((6 * CHUNK,), jnp.float32),
            pltpu.VMEM((CHUNK,), jnp.float32),
            pltpu.VMEM((CHUNK,), jnp.float32),
            pltpu.VMEM((CHUNK,), jnp.float32),
            pltpu.VMEM((CHUNK,), jnp.float32),
            pltpu.SemaphoreType.DMA,
            pltpu.SemaphoreType.DMA,
            pltpu.SemaphoreType.DMA,
            pltpu.SemaphoreType.DMA,
        ],
    )(xf, tbl)


def kernel(x, grid):
    xf = x.reshape(N * C * HW)
    gxy = jnp.stack([grid[..., 0].reshape(N, NCHUNKS, CHUNK),
                     grid[..., 1].reshape(N, NCHUNKS, CHUNK)],
                    axis=2).reshape(N * NCHUNKS * 2 * CHUNK)
    out = _grid_sample_sc(xf, gxy)
    return out.reshape(N, C, H, W)


# speedup vs baseline: 2.6284x; 1.0995x over previous
import jax
import jax.numpy as jnp
from jax import lax
from jax.experimental import pallas as pl
from jax.experimental.pallas import tpu as pltpu
from jax.experimental.pallas import tpu_sc as plsc

N, C, H, W = 4, 96, 224, 224
HW = H * W
L = 16
NC, NS = 2, 16
NW = NC * NS
CPAIRS = C // 2
TASKS_PER_TILE = (2 * CPAIRS) // NS
CHUNK_ROWS = 8
ROW_VECS = W // L
CHUNK = CHUNK_ROWS * W
CVECS = CHUNK // L
NCHUNKS = H // CHUNK_ROWS
NJOBS = N * NCHUNKS
JOBS_PER_TILE = -(-NJOBS // NW)
SX = float(W) / float(W - 1)
SY = float(H) / float(H - 1)


def _floor_i32(p):
    t = p.astype(jnp.int32)
    tf = t.astype(jnp.float32)
    return jnp.where(tf > p, t - 1, t)


def _tables_body(gxy_hbm, tbl_hbm, g_v, st_v):
    cid = lax.axis_index("c")
    sid = lax.axis_index("s")
    wid = sid * NC + cid

    def job_fn(q, carry):
        g = q * NW + wid

        @pl.when(g < NJOBS)
        def _():
            _job(g)

        return carry

    def _job(g):
        n = g // NCHUNKS
        ck = g % NCHUNKS
        pltpu.sync_copy(
            gxy_hbm.at[pl.ds((n * NCHUNKS + ck) * 2 * CHUNK, 2 * CHUNK)],
            g_v)

        def row_fn(ri, carry2):
            i = ck * CHUNK_ROWS + ri
            i_f = lax.convert_element_type(i, jnp.float32)

            @plsc.parallel_loop(0, ROW_VECS, unroll=2)
            def vec_fn(v):
                off = ri * W + v * L
                gxv = g_v[pl.ds(off, L)]
                gyv = g_v[pl.ds(CHUNK + off, L)]
                jb = lax.convert_element_type(v * L, jnp.float32)
                jf = lax.iota(jnp.int32, L).astype(jnp.float32) + jb
                px = (gxv + jf) * SX - 0.5
                py = (gyv + i_f) * SY - 0.5
                px = jnp.minimum(jnp.maximum(px, -2.0), float(W + 2))
                py = jnp.minimum(jnp.maximum(py, -2.0), float(H + 2))
                x0 = _floor_i32(px)
                y0 = _floor_i32(py)
                wx1 = px - x0.astype(jnp.float32)
                wx0 = 1.0 - wx1
                wy1 = py - y0.astype(jnp.float32)
                wy0 = 1.0 - wy1
                zero = jnp.zeros((L,), jnp.float32)
                wx0 = jnp.where((x0 >= 0) & (x0 <= W - 1), wx0, zero)
                wx1 = jnp.where((x0 >= -1) & (x0 <= W - 2), wx1, zero)
                wy0 = jnp.where((y0 >= 0) & (y0 <= H - 1), wy0, zero)
                wy1 = jnp.where((y0 >= -1) & (y0 <= H - 2), wy1, zero)
                xc0 = jnp.maximum(jnp.minimum(x0, W - 1), 0)
                xc1 = jnp.maximum(jnp.minimum(x0 + 1, W - 1), 0)
                yb0 = jnp.maximum(jnp.minimum(y0, H - 1), 0) * W
                yb1 = jnp.maximum(jnp.minimum(y0 + 1, H - 1), 0) * W
                ipk = (yb0 + xc0 + lax.shift_left(xc1 - xc0, 16)
                       + lax.shift_left(yb1 - yb0, 17))
                wxp = plsc.pack(wx0, wx1, format=plsc.PackFormat.INTERLEAVED)
                wyp = plsc.pack(wy0, wy1, format=plsc.PackFormat.INTERLEAVED)
                st_v[pl.ds(0 * CHUNK + off, L)] = plsc.bitcast(
                    ipk, jnp.float32)
                st_v[pl.ds(1 * CHUNK + off, L)] = plsc.bitcast(
                    wxp, jnp.float32)
                st_v[pl.ds(2 * CHUNK + off, L)] = plsc.bitcast(
                    wyp, jnp.float32)

            return carry2

        lax.fori_loop(0, CHUNK_ROWS, row_fn, 0)
        pltpu.sync_copy(
            st_v,
            tbl_hbm.at[pl.ds((n * NCHUNKS + ck) * 3 * CHUNK, 3 * CHUNK)])

    lax.fori_loop(0, JOBS_PER_TILE, job_fn, 0)


def _sample_body(x_hbm, tbl_hbm, out_hbm, img_v, ta_v, tb_v,
                 oa0_v, oa1_v, ob0_v, ob1_v, sta, stb, soa, sob):
    cid = lax.axis_index("c")
    sid = lax.axis_index("s")

    def t_src(n, ck):
        return tbl_hbm.at[pl.ds((n * NCHUNKS + ck) * 3 * CHUNK, 3 * CHUNK)]

    def compute_chunk(t_v, o0_v, o1_v):
        img1 = img_v.at[pl.ds(HW, HW)]

        @plsc.parallel_loop(0, CVECS, unroll=2)
        def vec_fn(p):
            off = p * L
            ipk = plsc.bitcast(t_v[pl.ds(0 * CHUNK + off, L)], jnp.int32)
            wxp = plsc.bitcast(t_v[pl.ds(1 * CHUNK + off, L)],
                               jnp.bfloat16)
            wyp = plsc.bitcast(t_v[pl.ds(2 * CHUNK + off, L)],
                               jnp.bfloat16)
            wx0, wx1 = plsc.unpack(wxp, format=plsc.PackFormat.INTERLEAVED)
            wy0, wy1 = plsc.unpack(wyp, format=plsc.PackFormat.INTERLEAVED)
            i00 = ipk & 0xFFFF
            dx = lax.shift_right_logical(ipk, 16) & 1
            dyw = lax.shift_right_logical(ipk, 17)
            i01 = i00 + dx
            i10 = i00 + dyw
            i11 = i01 + dyw
            w00 = wx0 * wy0
            w01 = wx1 * wy0
            w10 = wx0 * wy1
            w11 = wx1 * wy1
            a00 = plsc.load_gather(img_v, [i00])
            a01 = plsc.load_gather(img_v, [i01])
            a10 = plsc.load_gather(img_v, [i10])
            a11 = plsc.load_gather(img_v, [i11])
            o0_v[pl.ds(off, L)] = (w00 * a00 + w01 * a01
                                   + w10 * a10 + w11 * a11)
            b00 = plsc.load_gather(img1, [i00])
            b01 = plsc.load_gather(img1, [i01])
            b10 = plsc.load_gather(img1, [i10])
            b11 = plsc.load_gather(img1, [i11])
            o1_v[pl.ds(off, L)] = (w00 * b00 + w01 * b01
                                   + w10 * b10 + w11 * b11)

    def task_fn(t, carry):
        tt = sid * TASKS_PER_TILE + t
        n_local = tt // CPAIRS
        cp = tt % CPAIRS
        n = cid * 2 + n_local
        chan = n * C + cp * 2
        pltpu.sync_copy(x_hbm.at[pl.ds(chan * HW, 2 * HW)], img_v)
        pltpu.async_copy(t_src(n, 0), ta_v, sta)

        def out_dst(ck, ch):
            return out_hbm.at[pl.ds((chan + ch) * HW + ck * CHUNK, CHUNK)]

        def half(k, buf, t_v, t_next, o0_v, o1_v, st_this, st_next,
                 so_this):
            ck = k * 2 + buf
            nxt = jnp.minimum(ck + 1, NCHUNKS - 1)
            pltpu.async_copy(t_src(n, nxt), t_next, st_next)
            pltpu.make_async_copy(t_src(n, ck), t_v, st_this).wait()

            @pl.when(k > 0)
            def _():
                pltpu.make_async_copy(o0_v, out_dst(ck, 0), so_this).wait()
                pltpu.make_async_copy(o1_v, out_dst(ck, 1), so_this).wait()

            compute_chunk(t_v, o0_v, o1_v)
            pltpu.async_copy(o0_v, out_dst(ck, 0), so_this)
            pltpu.async_copy(o1_v, out_dst(ck, 1), so_this)

        def chunk_pair(k, carry2):
            half(k, 0, ta_v, tb_v, oa0_v, oa1_v, sta, stb, soa)
            half(k, 1, tb_v, ta_v, ob0_v, ob1_v, stb, sta, sob)
            return carry2

        lax.fori_loop(0, NCHUNKS // 2, chunk_pair, 0)
        pltpu.make_async_copy(t_src(n, NCHUNKS - 1), ta_v, sta).wait()
        pltpu.make_async_copy(oa0_v, out_dst(NCHUNKS - 2, 0), soa).wait()
        pltpu.make_async_copy(oa1_v, out_dst(NCHUNKS - 2, 1), soa).wait()
        pltpu.make_async_copy(ob0_v, out_dst(NCHUNKS - 1, 0), sob).wait()
        pltpu.make_async_copy(ob1_v, out_dst(NCHUNKS - 1, 1), sob).wait()
        return carry

    lax.fori_loop(0, TASKS_PER_TILE, task_fn, 0)


@jax.jit
def _grid_sample_sc(xf, gxy):
    mesh = plsc.VectorSubcoreMesh(core_axis_name="c", subcore_axis_name="s",
                                  num_cores=NC, num_subcores=NS)
    tbl = pl.kernel(
        _tables_body,
        out_type=jax.ShapeDtypeStruct((N * NCHUNKS * 3 * CHUNK,),
                                      jnp.float32),
        mesh=mesh,
        compiler_params=pltpu.CompilerParams(needs_layout_passes=False),
        scratch_types=[
            pltpu.VMEM((2 * CHUNK,), jnp.float32),
            pltpu.VMEM((3 * CHUNK,), jnp.float32),
        ],
    )(gxy)
    return pl.kernel(
        _sample_body,
        out_type=jax.ShapeDtypeStruct((N * C * HW,), jnp.float32),
        mesh=mesh,
        compiler_params=pltpu.CompilerParams(needs_layout_passes=False),
        scratch_types=[
            pltpu.VMEM((2 * HW,), jnp.float32),
            pltpu.VMEM((3 * CHUNK,), jnp.float32),
            pltpu.VMEM((3 * CHUNK,), jnp.float32),
            pltpu.VMEM((CHUNK,), jnp.float32),
            pltpu.VMEM((CHUNK,), jnp.float32),
            pltpu.VMEM((CHUNK,), jnp.float32),
            pltpu.VMEM((CHUNK,), jnp.float32),
            pltpu.SemaphoreType.DMA,
            pltpu.SemaphoreType.DMA,
            pltpu.SemaphoreType.DMA,
            pltpu.SemaphoreType.DMA,
        ],
    )(xf, tbl)


def kernel(x, grid):
    xf = x.reshape(N * C * HW)
    gxy = jnp.stack([grid[..., 0].reshape(N, NCHUNKS, CHUNK),
                     grid[..., 1].reshape(N, NCHUNKS, CHUNK)],
                    axis=2).reshape(N * NCHUNKS * 2 * CHUNK)
    out = _grid_sample_sc(xf, gxy)
    return out.reshape(N, C, H, W)
